# Initial kernel scaffold; baseline (speedup 1.0000x reference)
import jax
import jax.numpy as jnp
from jax.experimental import pallas as pl


def kernel(X, edge_index, edge_weight, W, Wf, Wj, Wc, Wo, Uf, Uj, Uc, Uo, bf, bj, bc, bo, h_init, c_init, lin_w, lin_b):
    raise NotImplementedError("write your pallas kernel here")


# speedup vs baseline: 55.0476x; 55.0476x over previous
import functools

import jax
import jax.numpy as jnp
from jax import lax
from jax.experimental import pallas as pl
from jax.experimental.pallas import tpu as pltpu
from jax.experimental.pallas import tpu_sc as plsc

_NC = 2
_NS = 16
_L = 16
_CH = 2048


def _proj_body(x_ref, w_ref, o_ref):
    o_ref[...] = jnp.dot(x_ref[...], w_ref[...],
                         preferred_element_type=jnp.float32,
                         precision=lax.Precision.HIGHEST)


def _project(Xf, W):
    R, F = Xf.shape
    H = W.shape[1]
    BR = 4000
    return pl.pallas_call(
        _proj_body,
        grid=(R // BR,),
        in_specs=[pl.BlockSpec((BR, F), lambda i: (i, 0)),
                  pl.BlockSpec((F, H), lambda i: (0, 0))],
        out_specs=pl.BlockSpec((BR, H), lambda i: (i, 0)),
        out_shape=jax.ShapeDtypeStruct((R, H), jnp.float32),
    )(Xf, W)


def _sc_spmm(XWf, srcp, dstp, wp, T, N, H):
    TT = T // _NC
    Epad = srcp.shape[0]
    ep_tile = Epad // _NS
    nchunk = ep_tile // _CH
    stripe = N // _NS

    mesh = plsc.VectorSubcoreMesh(core_axis_name="c", subcore_axis_name="s",
                                  num_cores=_NC, num_subcores=_NS)

    @functools.partial(
        pl.kernel,
        out_type=jax.ShapeDtypeStruct((T, N, H), jnp.float32),
        mesh=mesh,
        compiler_params=pltpu.CompilerParams(use_tc_tiling_on_sc=False),
        scratch_types=[
            pltpu.VMEM((_CH,), jnp.int32),
            pltpu.VMEM((_CH,), jnp.int32),
            pltpu.VMEM((_CH,), jnp.int32),
            pltpu.VMEM((_CH,), jnp.float32),
            pltpu.VMEM((_CH, H), jnp.float32),
            pltpu.VMEM((stripe, H), jnp.float32),
        ] + [pltpu.VMEM_SHARED((N, H), jnp.float32) for _ in range(TT)],
    )
    def spmm(xw_hbm, src_hbm, dst_hbm, w_hbm, out_hbm,
             src_buf, idx_buf, dst_buf, w_buf, rows_buf, zero_buf, *accs):
        c = lax.axis_index("c")
        s = lax.axis_index("s")
        row0 = s * stripe

        zvec = jnp.zeros((_L,), jnp.float32)

        def zero_body(r, _):
            zero_buf[r, :] = zvec
            return 0
        lax.fori_loop(0, stripe, zero_body, 0)
        for tt in range(TT):
            pltpu.sync_copy(zero_buf, accs[tt].at[pl.ds(row0, stripe), :])
        plsc.subcore_barrier()

        def chunk_body(ci, _):
            e0 = s * ep_tile + ci * _CH
            pltpu.sync_copy(src_hbm.at[pl.ds(e0, _CH)], src_buf)
            pltpu.sync_copy(dst_hbm.at[pl.ds(e0, _CH)], dst_buf)
            for tt in range(TT):
                t = c * TT + tt
                pltpu.sync_copy(w_hbm.at[t, pl.ds(e0, _CH)], w_buf)
                toff = t * N

                def off_body(g, _):
                    idx_buf[pl.ds(g * _L, _L)] = (
                        src_buf[pl.ds(g * _L, _L)] + toff)
                    return 0
                lax.fori_loop(0, _CH // _L, off_body, 0)

                pltpu.sync_copy(xw_hbm.at[idx_buf], rows_buf)

                def scale_body(g, _):
                    w16 = w_buf[pl.ds(g * _L, _L)]
                    base = g * _L
                    for l in range(_L):
                        wl = lax.broadcast(w16[l], (_L,))
                        rows_buf[base + l, :] = rows_buf[base + l, :] * wl
                    return 0
                lax.fori_loop(0, _CH // _L, scale_body, 0)

                pltpu.sync_copy(rows_buf, accs[tt].at[dst_buf], add=True)
            return 0
        lax.fori_loop(0, nchunk, chunk_body, 0)
        plsc.subcore_barrier()

        for tt in range(TT):
            t = c * TT + tt
            pltpu.sync_copy(accs[tt].at[pl.ds(row0, stripe), :],
                            out_hbm.at[t, pl.ds(row0, stripe), :])

    return spmm(XWf, srcp, dstp, wp)


def _lstm_body(axw_ref, wg_ref, ug_ref, bg_ref, h0_ref, c0_ref, lw_ref,
               lb_ref, out_ref):
    T, B, H = axw_ref.shape
    y_all = jnp.maximum(axw_ref[...], 0.0)
    h = jnp.broadcast_to(h0_ref[...], (B, H))
    c = jnp.broadcast_to(c0_ref[...], (B, H))
    wg = wg_ref[...]
    ug = ug_ref[...]
    bg = bg_ref[...]
    lw = lw_ref[...]
    lb = lb_ref[0, 0]
    for t in range(T):
        g = (jnp.dot(y_all[t], wg, preferred_element_type=jnp.float32,
                     precision=lax.Precision.HIGHEST)
             + jnp.dot(h, ug, preferred_element_type=jnp.float32,
                       precision=lax.Precision.HIGHEST) + bg)
        f = jax.nn.sigmoid(g[:, 0:H])
        j = jax.nn.sigmoid(g[:, H:2 * H])
        o = jax.nn.sigmoid(g[:, 2 * H:3 * H])
        ct = jax.nn.sigmoid(g[:, 3 * H:4 * H])
        c = j * ct + f * c
        h = o * jnp.tanh(c)
        z = jnp.dot(h, lw, preferred_element_type=jnp.float32,
                    precision=lax.Precision.HIGHEST) + lb
        out_ref[t, :] = z[:, 0]


def _lstm(AXW, Wg, Ug, bg, h0, c0, lwT, lb):
    T, N, H = AXW.shape
    B = 2048
    return pl.pallas_call(
        _lstm_body,
        grid=(pl.cdiv(N, B),),
        in_specs=[
            pl.BlockSpec((T, B, H), lambda i: (0, i, 0)),
            pl.BlockSpec((H, 4 * H), lambda i: (0, 0)),
            pl.BlockSpec((H, 4 * H), lambda i: (0, 0)),
            pl.BlockSpec((1, 4 * H), lambda i: (0, 0)),
            pl.BlockSpec((1, H), lambda i: (0, 0)),
            pl.BlockSpec((1, H), lambda i: (0, 0)),
            pl.BlockSpec((H, 1), lambda i: (0, 0)),
            pl.BlockSpec((1, 1), lambda i: (0, 0)),
        ],
        out_specs=pl.BlockSpec((T, B), lambda i: (0, i)),
        out_shape=jax.ShapeDtypeStruct((T, N), jnp.float32),
    )(AXW, Wg, Ug, bg, h0, c0, lwT, lb)


def kernel(X, edge_index, edge_weight, W, Wf, Wj, Wc, Wo, Uf, Uj, Uc, Uo,
           bf, bj, bc, bo, h_init, c_init, lin_w, lin_b):
    T, N, F = X.shape
    H = W.shape[1]
    E = edge_index.shape[1]

    XWf = _project(X.reshape(T * N, F), W)

    chunk_grain = _NS * _CH
    Epad = ((E + chunk_grain - 1) // chunk_grain) * chunk_grain
    pad = Epad - E
    srcp = jnp.concatenate([edge_index[0], jnp.zeros((pad,), jnp.int32)])
    dstp = jnp.concatenate([edge_index[1], jnp.zeros((pad,), jnp.int32)])
    wp = jnp.concatenate(
        [edge_weight, jnp.zeros((T, pad), jnp.float32)], axis=1)

    AXW = _sc_spmm(XWf, srcp, dstp, wp, T, N, H)

    Wg = jnp.concatenate([Wf, Wj, Wo, Wc], axis=1)
    Ug = jnp.concatenate([Uf, Uj, Uo, Uc], axis=1)
    bg = jnp.concatenate([bf, bj, bo, bc]).reshape(1, 4 * H)
    return _lstm(AXW, Wg, Ug, bg, h_init.reshape(1, H), c_init.reshape(1, H),
                 lin_w.reshape(H, 1), lin_b.reshape(1, 1))
